# baseline (speedup 1.0000x reference)
import functools

import jax
import jax.numpy as jnp
from jax import lax
from jax.experimental import pallas as pl
from jax.experimental.pallas import tpu as pltpu
from jax.experimental.pallas import tpu_sc as plsc

_D = 64
_SCALE = 8.0
_CG = 128
_KG = 4
_C = _CG * _KG


def _gather_scale(x2d, table):
    nrows, cg = x2d.shape
    B = nrows * cg
    info = plsc.get_sparse_core_info()
    NW = info.num_cores * info.num_subcores
    b_per_w = B // NW
    nchunks = b_per_w // _C

    mesh = plsc.VectorSubcoreMesh(core_axis_name="c", subcore_axis_name="s")

    @functools.partial(
        pl.kernel,
        mesh=mesh,
        compiler_params=pltpu.CompilerParams(use_tc_tiling_on_sc=False),
        out_type=jax.ShapeDtypeStruct((B, _D), jnp.float32),
        scratch_types=[
            pltpu.VMEM((_KG, _CG), jnp.int32),
            pltpu.VMEM((_C, _D), jnp.float32),
            pltpu.SemaphoreType.DMA,
        ],
    )
    def k(x_hbm, tab_hbm, out_hbm, idx_v, rows_v, sem):
        wid = lax.axis_index("s") * info.num_cores + lax.axis_index("c")
        wrow = wid * (b_per_w // _CG)

        def chunk_body(g, carry):
            row0 = wrow + g * _KG
            base = wid * b_per_w + g * _C
            pltpu.sync_copy(x_hbm.at[pl.ds(row0, _KG)], idx_v)
            cps = [
                pltpu.async_copy(
                    tab_hbm.at[idx_v.at[j]],
                    rows_v.at[pl.ds(j * _CG, _CG)],
                    sem,
                )
                for j in range(_KG)
            ]
            for cp in cps:
                cp.wait()

            def row_body(i, c2):
                for q in range(_D // 16):
                    sl = pl.ds(q * 16, 16)
                    rows_v[i, sl] = rows_v[i, sl] * _SCALE
                return c2

            lax.fori_loop(0, _C, row_body, 0, unroll=4)
            pltpu.sync_copy(rows_v, out_hbm.at[pl.ds(base, _C)])
            return carry

        lax.fori_loop(0, nchunks, chunk_body, 0)

    return k(x2d, table)


def kernel(x, emb_weight):
    b, s = x.shape
    B = b * s
    x2d = x.reshape(B // _CG, _CG).astype(jnp.int32)
    out = _gather_scale(x2d, emb_weight)
    return out.reshape(b, s, _D)
